# baseline (speedup 1.0000x reference)
import functools

import jax
import jax.numpy as jnp
from jax import lax
from jax.experimental import pallas as pl
from jax.experimental.pallas import tpu as pltpu
from jax.experimental.pallas import tpu_sc as plsc

N_NODE = 4096
N_LOC = 3
N_T = 4
BATCH = 65536
TAU_MAX = 50.0
PLANE = N_NODE * N_NODE

_INFO = plsc.get_sparse_core_info()
NC = _INFO.num_cores
NS = _INFO.num_subcores
L = _INFO.num_lanes
NW = NC * NS
BPW = BATCH // NW
NSUB = 4
CHUNK = BPW // NSUB
CPS = CHUNK // L


@functools.partial(
    pl.kernel,
    mesh=plsc.VectorSubcoreMesh(core_axis_name="c", subcore_axis_name="s"),
    out_type=jax.ShapeDtypeStruct((BATCH,), jnp.float32),
    scratch_types=[
        pltpu.VMEM((BPW,), jnp.float32),
        pltpu.VMEM((BPW,), jnp.int32),
        pltpu.VMEM((BPW,), jnp.int32),
        pltpu.VMEM((NSUB * N_LOC * CHUNK,), jnp.int32),
        pltpu.VMEM((NSUB * N_LOC * CHUNK,), jnp.float32),
        pltpu.VMEM((BPW,), jnp.float32),
        pltpu.VMEM(((N_T * N_LOC + N_T) * L,), jnp.float32),
        pltpu.VMEM(((N_T * N_LOC + N_T) * L,), jnp.int32),
        pltpu.SemaphoreType.DMA,
        pltpu.SemaphoreType.DMA,
        pltpu.SemaphoreType.DMA,
        pltpu.SemaphoreType.DMA,
        pltpu.SemaphoreType.DMA,
        pltpu.SemaphoreType.DMA,
        pltpu.SemaphoreType.DMA,
        pltpu.SemaphoreType.DMA,
    ],
)
def _sc_cheb_filter(bv_h, dt_h, xi_h, yi_h, ws_h, out_h,
                    dt_v, xi_v, yi_v, idx_v, g_v, o_v, ws_v, idxw_v,
                    sem_dt, sem_xi, sem_yi, sem_ws, sg0, sg1, sg2, sg3):
    sg = (sg0, sg1, sg2, sg3)
    wid = lax.axis_index("s") * NC + lax.axis_index("c")
    base = wid * BPW
    cp_dt = pltpu.async_copy(dt_h.at[pl.ds(base, BPW)], dt_v, sem_dt)
    cp_xi = pltpu.async_copy(xi_h.at[pl.ds(base, BPW)], xi_v, sem_xi)
    cp_yi = pltpu.async_copy(yi_h.at[pl.ds(base, BPW)], yi_v, sem_yi)
    for m in range(N_T * N_LOC + N_T):
        idxw_v[pl.ds(m * L, L)] = jnp.zeros((L,), jnp.int32) + m
    cp_ws = pltpu.async_copy(ws_h.at[idxw_v], ws_v, sem_ws)
    cp_xi.wait()
    cp_yi.wait()

    gathers = []
    for k in range(NSUB):
        @plsc.parallel_loop(0, CPS, unroll=4)
        def build_idx(c, _k=k):
            s = pl.ds(_k * CHUNK + c * L, L)
            r = yi_v[s]
            cc = xi_v[s]
            fi = ((r >> 3) << 15) + ((cc >> 7) << 10) + ((r & 7) << 7) + (cc & 127)
            kb = _k * N_LOC * CHUNK
            idx_v[pl.ds(kb + c * L, L)] = fi
            idx_v[pl.ds(kb + CHUNK + c * L, L)] = fi + PLANE
            idx_v[pl.ds(kb + 2 * CHUNK + c * L, L)] = fi + 2 * PLANE

        kb = k * N_LOC * CHUNK
        gathers.append(pltpu.async_copy(
            bv_h.at[idx_v.at[pl.ds(kb, N_LOC * CHUNK)]],
            g_v.at[pl.ds(kb, N_LOC * CHUNK)], sg[k]))

    cp_dt.wait()
    cp_ws.wait()
    w = [ws_v[pl.ds(m * L, L)] for m in range(N_T * N_LOC)]
    ns = []
    for i in range(N_T):
        sig = ws_v[pl.ds((N_T * N_LOC + i) * L, L)]
        ns.append(-1.0 / (2.0 * sig * sig))

    for k in range(NSUB):
        gathers[k].wait()

        @plsc.parallel_loop(0, CPS, unroll=2)
        def combine(c, _k=k):
            s = pl.ds(_k * CHUNK + c * L, L)
            dt = dt_v[s]
            dt2 = dt * dt
            es = [jnp.exp(dt2 * ns[i]) for i in range(N_T)]
            res = jnp.zeros((L,), jnp.float32)
            for j in range(N_LOC):
                a = w[j] * es[0]
                for i in range(1, N_T):
                    a = a + w[i * N_LOC + j] * es[i]
                res = res + a * g_v[pl.ds(_k * N_LOC * CHUNK + j * CHUNK + c * L, L)]
            o_v[s] = jnp.where(dt <= TAU_MAX, res, 0.0)

    pltpu.sync_copy(o_v, out_h.at[pl.ds(base, BPW)])


def kernel(x, y, xi, yi, B, mask_B, weights, sigmas):
    del mask_B
    dt = x[:, 0] - y[:, 0]
    xi32 = xi.astype(jnp.int32)
    yi32 = yi.astype(jnp.int32)
    bflat = (B.reshape(N_LOC, N_NODE // 8, 8, N_NODE // 128, 128)
             .transpose(0, 1, 3, 2, 4)
             .reshape(N_LOC * PLANE))
    ws = jnp.concatenate([weights.astype(jnp.float32).reshape(N_T * N_LOC),
                          sigmas.astype(jnp.float32)])
    return _sc_cheb_filter(bflat, dt, xi32, yi32, ws)


# speedup vs baseline: 2.1566x; 2.1566x over previous
import functools

import jax
import jax.numpy as jnp
from jax import lax
from jax.experimental import pallas as pl
from jax.experimental.pallas import tpu as pltpu
from jax.experimental.pallas import tpu_sc as plsc

N_NODE = 4096
N_LOC = 3
N_T = 4
BATCH = 65536
TAU_MAX = 50.0
PLANE = N_NODE * N_NODE

_INFO = plsc.get_sparse_core_info()
NC = _INFO.num_cores
NS = _INFO.num_subcores
L = _INFO.num_lanes
NW = NC * NS
BPW = BATCH // NW
NSUB = 4
CHUNK = BPW // NSUB
CPS = CHUNK // L


@functools.partial(
    pl.kernel,
    mesh=plsc.VectorSubcoreMesh(core_axis_name="c", subcore_axis_name="s"),
    out_type=jax.ShapeDtypeStruct((BATCH,), jnp.float32),
    scratch_types=[
        pltpu.VMEM((BPW,), jnp.float32),
        pltpu.VMEM((BPW,), jnp.int32),
        pltpu.VMEM((BPW,), jnp.int32),
        pltpu.VMEM((NSUB * N_LOC * CHUNK,), jnp.int32),
        pltpu.VMEM((NSUB * N_LOC * CHUNK,), jnp.float32),
        pltpu.VMEM((BPW,), jnp.float32),
        pltpu.VMEM((N_T * N_LOC + N_T, L), jnp.float32),
        pltpu.SemaphoreType.DMA,
        pltpu.SemaphoreType.DMA,
        pltpu.SemaphoreType.DMA,
        pltpu.SemaphoreType.DMA,
        pltpu.SemaphoreType.DMA,
        pltpu.SemaphoreType.DMA,
        pltpu.SemaphoreType.DMA,
        pltpu.SemaphoreType.DMA,
    ],
)
def _sc_cheb_filter(bv_h, dt_h, xi_h, yi_h, ws_h, out_h,
                    dt_v, xi_v, yi_v, idx_v, g_v, o_v, ws_v,
                    sem_dt, sem_xi, sem_yi, sem_ws, sg0, sg1, sg2, sg3):
    sg = (sg0, sg1, sg2, sg3)
    wid = lax.axis_index("s") * NC + lax.axis_index("c")
    base = wid * BPW
    cp_dt = pltpu.async_copy(dt_h.at[pl.ds(base, BPW)], dt_v, sem_dt)
    cp_xi = pltpu.async_copy(xi_h.at[pl.ds(base, BPW)], xi_v, sem_xi)
    cp_yi = pltpu.async_copy(yi_h.at[pl.ds(base, BPW)], yi_v, sem_yi)
    cp_ws = pltpu.async_copy(ws_h, ws_v, sem_ws)
    cp_xi.wait()
    cp_yi.wait()

    gathers = []
    for k in range(NSUB):
        @plsc.parallel_loop(0, CPS, unroll=4)
        def build_idx(c, _k=k):
            s = pl.ds(_k * CHUNK + c * L, L)
            r = yi_v[s]
            cc = xi_v[s]
            fi = ((r >> 3) << 15) + ((cc >> 7) << 10) + ((r & 7) << 7) + (cc & 127)
            kb = _k * N_LOC * CHUNK
            idx_v[pl.ds(kb + c * L, L)] = fi
            idx_v[pl.ds(kb + CHUNK + c * L, L)] = fi + PLANE
            idx_v[pl.ds(kb + 2 * CHUNK + c * L, L)] = fi + 2 * PLANE

        kb = k * N_LOC * CHUNK
        gathers.append(pltpu.async_copy(
            bv_h.at[idx_v.at[pl.ds(kb, N_LOC * CHUNK)]],
            g_v.at[pl.ds(kb, N_LOC * CHUNK)], sg[k]))

    cp_dt.wait()
    cp_ws.wait()
    w = [ws_v[m] for m in range(N_T * N_LOC)]
    ns = []
    for i in range(N_T):
        sig = ws_v[N_T * N_LOC + i]
        ns.append(-1.0 / (2.0 * sig * sig))

    for k in range(NSUB):
        gathers[k].wait()

        @plsc.parallel_loop(0, CPS, unroll=2)
        def combine(c, _k=k):
            s = pl.ds(_k * CHUNK + c * L, L)
            dt = dt_v[s]
            dt2 = dt * dt
            es = [jnp.exp(dt2 * ns[i]) for i in range(N_T)]
            res = jnp.zeros((L,), jnp.float32)
            for j in range(N_LOC):
                a = w[j] * es[0]
                for i in range(1, N_T):
                    a = a + w[i * N_LOC + j] * es[i]
                res = res + a * g_v[pl.ds(_k * N_LOC * CHUNK + j * CHUNK + c * L, L)]
            o_v[s] = jnp.where(dt <= TAU_MAX, res, 0.0)

    pltpu.sync_copy(o_v, out_h.at[pl.ds(base, BPW)])


def kernel(x, y, xi, yi, B, mask_B, weights, sigmas):
    del mask_B
    dt = x[:, 0] - y[:, 0]
    xi32 = xi.astype(jnp.int32)
    yi32 = yi.astype(jnp.int32)
    bflat = (B.reshape(N_LOC, N_NODE // 8, 8, N_NODE // 128, 128)
             .transpose(0, 1, 3, 2, 4)
             .reshape(N_LOC * PLANE))
    ws = jnp.broadcast_to(
        jnp.concatenate([weights.astype(jnp.float32).reshape(N_T * N_LOC),
                         sigmas.astype(jnp.float32)])[:, None],
        (N_T * N_LOC + N_T, L))
    return _sc_cheb_filter(bflat, dt, xi32, yi32, ws)
